# Initial kernel scaffold; baseline (speedup 1.0000x reference)
import jax
import jax.numpy as jnp
from jax.experimental import pallas as pl


def kernel(x, edge_index, batch, W0a, b0a, W0b, b0b, W1a, b1a, W1b, b1b, W2a, b2a, W2b, b2b, ln_g, ln_b, bn_g, bn_b, bn_m, bn_v):
    raise NotImplementedError("write your pallas kernel here")


# speedup vs baseline: 2.6156x; 2.6156x over previous
import functools

import jax
import jax.numpy as jnp
from jax import lax
from jax.experimental import pallas as pl
from jax.experimental.pallas import tpu as pltpu
from jax.experimental.pallas import tpu_sc as plsc

N = 10000
D = 128
E = 320000
G = 64

NC = 2
NS = 16
NW = NC * NS

CHUNK = 128
EPW_CHUNKS = 80
EPW = CHUNK * EPW_CHUNKS
E_PAD = NW * EPW
AGG_ROWS = 10112
ZROWS = AGG_ROWS // NS


@functools.partial(
    pl.kernel,
    out_type=jax.ShapeDtypeStruct((NC, AGG_ROWS, D), jnp.float32),
    mesh=plsc.VectorSubcoreMesh(core_axis_name="c", subcore_axis_name="s"),
    scratch_types=[
        pltpu.VMEM((EPW_CHUNKS, CHUNK), jnp.int32),
        pltpu.VMEM((EPW_CHUNKS, CHUNK), jnp.int32),
        pltpu.VMEM((CHUNK, D), jnp.float32),
        pltpu.VMEM_SHARED((AGG_ROWS, D), jnp.float32),
        pltpu.SemaphoreType.DMA,
    ],
)
def _sc_scatter(h_hbm, src_hbm, dst_hbm, zero_hbm, out_hbm,
                src_v, dst_v, rows_v, agg_sh, sem):
    c = lax.axis_index("c")
    s = lax.axis_index("s")
    wid = c * NS + s

    pltpu.sync_copy(zero_hbm, agg_sh.at[pl.ds(s * ZROWS, ZROWS)])
    pltpu.sync_copy(src_hbm.at[wid], src_v)
    pltpu.sync_copy(dst_hbm.at[wid], dst_v)
    plsc.subcore_barrier()

    def step(j, carry):
        pltpu.async_copy(h_hbm.at[src_v.at[j]], rows_v, sem).wait()
        pltpu.sync_copy(rows_v, agg_sh.at[dst_v.at[j]], add=True)
        return carry

    lax.fori_loop(0, EPW_CHUNKS, step, 0)
    plsc.subcore_barrier()

    pltpu.sync_copy(agg_sh.at[pl.ds(s * ZROWS, ZROWS)],
                    out_hbm.at[c, pl.ds(s * ZROWS, ZROWS)])


R = 1000
NB = N // R


def _leaky(v):
    return jnp.where(v >= 0, v, 0.2 * v)


def _mlp_block(h_ref, p_ref, wa_ref, ba_ref, wb_ref, bb_ref):
    m = h_ref[...] + p_ref[0] + p_ref[1]
    y = _leaky(jnp.dot(m, wa_ref[...], preferred_element_type=jnp.float32)
               + ba_ref[...])
    return _leaky(jnp.dot(y, wb_ref[...], preferred_element_type=jnp.float32)
                  + bb_ref[...])


def _mlp_body(h_ref, p_ref, wa_ref, ba_ref, wb_ref, bb_ref, o_ref):
    o_ref[...] = _mlp_block(h_ref, p_ref, wa_ref, ba_ref, wb_ref, bb_ref)


def _mlp(h, parts, wa, ba, wb, bb):
    return pl.pallas_call(
        _mlp_body,
        grid=(NB,),
        in_specs=[
            pl.BlockSpec((R, D), lambda i: (i, 0)),
            pl.BlockSpec((NC, R, D), lambda i: (0, i, 0)),
            pl.BlockSpec((D, D), lambda i: (0, 0)),
            pl.BlockSpec((1, D), lambda i: (0, 0)),
            pl.BlockSpec((D, D), lambda i: (0, 0)),
            pl.BlockSpec((1, D), lambda i: (0, 0)),
        ],
        out_specs=pl.BlockSpec((R, D), lambda i: (i, 0)),
        out_shape=jax.ShapeDtypeStruct((N, D), jnp.float32),
    )(h, parts, wa, ba, wb, bb)


def _final_body(h_ref, p_ref, wa_ref, ba_ref, wb_ref, bb_ref,
                lng_ref, lnb_ref, batch_ref,
                bng_ref, bnb_ref, bnm_ref, bnv_ref,
                node_ref, graph_ref, acc_ref):
    i = pl.program_id(0)
    z = _mlp_block(h_ref, p_ref, wa_ref, ba_ref, wb_ref, bb_ref)

    mu = jnp.mean(z, axis=-1, keepdims=True)
    var = jnp.mean((z - mu) ** 2, axis=-1, keepdims=True)
    node = (z - mu) * lax.rsqrt(var + 1e-5) * lng_ref[...] + lnb_ref[...]
    node_ref[...] = node

    b = batch_ref[0]
    iota_g = lax.broadcasted_iota(jnp.int32, (G, R), 0)
    onehot = (iota_g == b).astype(jnp.float32)
    contrib = lax.dot_general(onehot, node, (((1,), (0,)), ((), ())),
                              preferred_element_type=jnp.float32)

    @pl.when(i == 0)
    def _():
        acc_ref[...] = jnp.zeros_like(acc_ref)

    acc_ref[...] += contrib

    @pl.when(i == NB - 1)
    def _():
        inv = lax.rsqrt(bnv_ref[...] + 1e-5)
        graph_ref[...] = ((acc_ref[...] - bnm_ref[...]) * inv * bng_ref[...]
                          + bnb_ref[...])


def _mlp_final(h, parts, wa, ba, wb, bb, ln_g, ln_b, batch3, bn_g, bn_b, bn_m, bn_v):
    vec = pl.BlockSpec((1, D), lambda i: (0, 0))
    return pl.pallas_call(
        _final_body,
        grid=(NB,),
        in_specs=[
            pl.BlockSpec((R, D), lambda i: (i, 0)),
            pl.BlockSpec((NC, R, D), lambda i: (0, i, 0)),
            pl.BlockSpec((D, D), lambda i: (0, 0)),
            vec,
            pl.BlockSpec((D, D), lambda i: (0, 0)),
            vec,
            vec,
            vec,
            pl.BlockSpec((1, 1, R), lambda i: (i, 0, 0)),
            vec,
            vec,
            vec,
            vec,
        ],
        out_specs=[
            pl.BlockSpec((R, D), lambda i: (i, 0)),
            pl.BlockSpec((G, D), lambda i: (0, 0)),
        ],
        out_shape=[
            jax.ShapeDtypeStruct((N, D), jnp.float32),
            jax.ShapeDtypeStruct((G, D), jnp.float32),
        ],
        scratch_shapes=[pltpu.VMEM((G, D), jnp.float32)],
    )(h, parts, wa, ba, wb, bb, ln_g, ln_b, batch3, bn_g, bn_b, bn_m, bn_v)


def kernel(x, edge_index, batch, W0a, b0a, W0b, b0b, W1a, b1a, W1b, b1b,
           W2a, b2a, W2b, b2b, ln_g, ln_b, bn_g, bn_b, bn_m, bn_v):
    pad = E_PAD - E
    src_p = jnp.concatenate(
        [edge_index[0], jnp.zeros((pad,), jnp.int32)]).reshape(NW, EPW_CHUNKS, CHUNK)
    dst_p = jnp.concatenate(
        [edge_index[1], jnp.full((pad,), N, jnp.int32)]).reshape(NW, EPW_CHUNKS, CHUNK)
    zero = jnp.zeros((ZROWS, D), jnp.float32)
    batch3 = batch.reshape(NB, 1, R)

    r2 = lambda v: v.reshape(1, D)
    layers = [(W0a, r2(b0a), W0b, r2(b0b)),
              (W1a, r2(b1a), W1b, r2(b1b)),
              (W2a, r2(b2a), W2b, r2(b2b))]

    h = x
    for l, (wa, ba, wb, bb) in enumerate(layers):
        parts = _sc_scatter(h, src_p, dst_p, zero)
        if l < 2:
            h = _mlp(h, parts, wa, ba, wb, bb)
        else:
            node, graph = _mlp_final(h, parts, wa, ba, wb, bb,
                                     r2(ln_g), r2(ln_b), batch3,
                                     r2(bn_g), r2(bn_b), r2(bn_m), r2(bn_v))
    return node, graph
